# Initial kernel scaffold; baseline (speedup 1.0000x reference)
import jax
import jax.numpy as jnp
from jax.experimental import pallas as pl


def kernel(x, token_table, pos_table):
    raise NotImplementedError("write your pallas kernel here")


# speedup vs baseline: 3.1658x; 3.1658x over previous
import jax
import jax.numpy as jnp
from jax import lax
from jax.experimental import pallas as pl
from jax.experimental.pallas import tpu as pltpu
from jax.experimental.pallas import tpu_sc as plsc

VOCAB = 100000
EMBED_DIM = 64
BATCH = 4096
SEQ = 200

NC = 2
NS = 16
NW = NC * NS
SEQ_PER_W = BATCH // NW
LANES = 16
VPR = EMBED_DIM // LANES


def _body(x_ref, tok_ref, pos_ref, out_ref, idx_v, pos_v, rows_v, gsem):
    wid = lax.axis_index("s") * NC + lax.axis_index("c")
    base_row = wid * SEQ_PER_W * SEQ

    pltpu.sync_copy(x_ref.at[pl.ds(base_row, SEQ_PER_W * SEQ)], idx_v)
    pltpu.sync_copy(pos_ref.at[pl.ds(0, SEQ)], pos_v)

    def per_seq(i, carry):
        off = i * SEQ
        cp0 = pltpu.async_copy(
            tok_ref.at[idx_v.at[pl.ds(off, 128)]],
            rows_v.at[pl.ds(0, 128)], gsem)
        cp1 = pltpu.async_copy(
            tok_ref.at[idx_v.at[pl.ds(off + 128, SEQ - 128)]],
            rows_v.at[pl.ds(128, SEQ - 128)], gsem)
        cp0.wait()
        cp1.wait()

        def add_row(r, c2):
            for d in range(VPR):
                sl = pl.ds(d * LANES, LANES)
                rows_v[r, sl] = rows_v[r, sl] + pos_v[r, sl]
            return c2

        lax.fori_loop(0, SEQ, add_row, 0)
        pltpu.sync_copy(rows_v, out_ref.at[pl.ds(base_row + off, SEQ)])
        return carry

    lax.fori_loop(0, SEQ_PER_W, per_seq, 0)


def kernel(x, token_table, pos_table):
    x_flat = x.reshape(BATCH * SEQ).astype(jnp.int32)
    mesh = plsc.VectorSubcoreMesh(core_axis_name="c", subcore_axis_name="s")
    k = pl.kernel(
        _body,
        mesh=mesh,
        compiler_params=pltpu.CompilerParams(use_tc_tiling_on_sc=False),
        out_type=jax.ShapeDtypeStruct((BATCH * SEQ, EMBED_DIM), jnp.float32),
        scratch_types=[
            pltpu.VMEM((SEQ_PER_W * SEQ,), jnp.int32),
            pltpu.VMEM((SEQ, EMBED_DIM), jnp.float32),
            pltpu.VMEM((SEQ, EMBED_DIM), jnp.float32),
            pltpu.SemaphoreType.DMA,
        ],
    )
    out = k(x_flat, token_table, pos_table)
    return out.reshape(BATCH, SEQ, EMBED_DIM)


# speedup vs baseline: 4.0088x; 1.2663x over previous
import jax
import jax.numpy as jnp
from jax import lax
from jax.experimental import pallas as pl
from jax.experimental.pallas import tpu as pltpu
from jax.experimental.pallas import tpu_sc as plsc

VOCAB = 100000
EMBED_DIM = 64
BATCH = 4096
SEQ = 200

NC = 2
NS = 16
NW = NC * NS
SEQ_PER_W = BATCH // NW
LANES = 16
VPR = EMBED_DIM // LANES
NBUF = 4
PD = 2
CHUNK = 128


def _body(x_ref, tok_ref, pos_ref, out_ref, idx_v, pos_v,
          rows0, rows1, rows2, rows3,
          g0, g1, g2, g3, s0, s1, s2, s3):
    bufs = (rows0, rows1, rows2, rows3)
    gsems = (g0, g1, g2, g3)
    ssems = (s0, s1, s2, s3)

    wid = lax.axis_index("s") * NC + lax.axis_index("c")
    base_row = wid * SEQ_PER_W * SEQ

    pltpu.sync_copy(x_ref.at[pl.ds(base_row, SEQ_PER_W * SEQ)], idx_v)
    pltpu.sync_copy(pos_ref.at[pl.ds(0, SEQ)], pos_v)

    def issue_gather(i, rows, gsem):
        off = i * SEQ
        pltpu.async_copy(tok_ref.at[idx_v.at[pl.ds(off, CHUNK)]],
                         rows.at[pl.ds(0, CHUNK)], gsem)
        pltpu.async_copy(tok_ref.at[idx_v.at[pl.ds(off + CHUNK, SEQ - CHUNK)]],
                         rows.at[pl.ds(CHUNK, SEQ - CHUNK)], gsem)

    def drain_gather(rows, gsem):
        pltpu.make_async_copy(tok_ref.at[pl.ds(0, SEQ)], rows, gsem).wait()

    def drain_store(rows, ssem):
        pltpu.make_async_copy(rows, out_ref.at[pl.ds(0, SEQ)], ssem).wait()

    for b in range(PD):
        issue_gather(b, bufs[b], gsems[b])

    def quad(t, carry):
        i4 = t * NBUF
        for b in range(NBUF):
            i = i4 + b
            rows, gsem, ssem = bufs[b], gsems[b], ssems[b]
            drain_gather(rows, gsem)

            j = (b + PD) % NBUF
            nxt = i + PD

            @pl.when(nxt < SEQ_PER_W)
            def _():
                @pl.when(nxt >= NBUF)
                def _():
                    drain_store(bufs[j], ssems[j])
                issue_gather(nxt, bufs[j], gsems[j])

            @plsc.parallel_loop(0, SEQ)
            def _(r):
                for d in range(VPR):
                    sl = pl.ds(d * LANES, LANES)
                    rows[r, sl] = rows[r, sl] + pos_v[r, sl]

            pltpu.async_copy(rows, out_ref.at[pl.ds(base_row + i * SEQ, SEQ)],
                             ssem)
        return carry

    lax.fori_loop(0, SEQ_PER_W // NBUF, quad, 0)

    for b in range(NBUF):
        drain_store(bufs[b], ssems[b])


def kernel(x, token_table, pos_table):
    x_flat = x.reshape(BATCH * SEQ).astype(jnp.int32)
    mesh = plsc.VectorSubcoreMesh(core_axis_name="c", subcore_axis_name="s")
    k = pl.kernel(
        _body,
        mesh=mesh,
        compiler_params=pltpu.CompilerParams(use_tc_tiling_on_sc=False),
        out_type=jax.ShapeDtypeStruct((BATCH * SEQ, EMBED_DIM), jnp.float32),
        scratch_types=[
            pltpu.VMEM((SEQ_PER_W * SEQ,), jnp.int32),
            pltpu.VMEM((SEQ, EMBED_DIM), jnp.float32),
        ] + [pltpu.VMEM((SEQ, EMBED_DIM), jnp.float32) for _ in range(NBUF)]
          + [pltpu.SemaphoreType.DMA for _ in range(2 * NBUF)],
    )
    out = k(x_flat, token_table, pos_table)
    return out.reshape(BATCH, SEQ, EMBED_DIM)
